# Initial kernel scaffold; baseline (speedup 1.0000x reference)
import jax
import jax.numpy as jnp
from jax.experimental import pallas as pl


def kernel(time, negative_lambdas, quantities, top_k, min_tokens_to_keep):
    raise NotImplementedError("write your pallas kernel here")


# speedup vs baseline: 2.3982x; 2.3982x over previous
import jax
import jax.numpy as jnp
from jax.experimental import pallas as pl
from jax.experimental.pallas import tpu as pltpu

N = 32768
H = 16
D = 64
S = 8
L = N // S
K = min(N, max(4096, 16))
CH = 512
NCH = N // CH


def _rel_body(t_ref, nll_ref, q_ref, rel_ref):
    q = q_ref[0]
    mag2 = jnp.sum(q * q, axis=-1)
    rel_ref[0, 0] = jnp.sqrt(mag2) * jnp.exp(nll_ref[0, 0] * t_ref[0, 0])


def _sel_mm_body(rels_ref, relc_ref, t_ref, nl_ref, q_ref, out_ref, sc_ref):
    c = pl.program_id(0)

    @pl.when(c == 0)
    def _search():
        rel = rels_ref[...]
        relbits = jax.lax.bitcast_convert_type(rel, jnp.int32)

        def vsearch(i, lo):
            cand = lo | (jnp.int32(1) << (jnp.int32(30) - i))
            cnt = jnp.sum((relbits >= cand).astype(jnp.int32))
            return jnp.where(cnt >= K, cand, lo)

        taubits = jax.lax.fori_loop(0, 31, vsearch, jnp.int32(0))

        cnt_gt = jnp.sum((relbits > taubits).astype(jnp.int32))
        need = K - cnt_gt
        eq = relbits == taubits
        idx = (jax.lax.broadcasted_iota(jnp.int32, (S, L), 0) * L
               + jax.lax.broadcasted_iota(jnp.int32, (S, L), 1))

        def isearch(i, m):
            cand = m | (jnp.int32(1) << (jnp.int32(14) - i))
            cnt = jnp.sum(jnp.where(eq & (idx < cand), 1, 0))
            return jnp.where(cnt < need, cand, m)

        mlow = jax.lax.fori_loop(0, 15, isearch, jnp.int32(0))
        sc_ref[0] = taubits
        sc_ref[1] = mlow

    tau = sc_ref[0]
    mlow = sc_ref[1]
    bits = jax.lax.bitcast_convert_type(relc_ref[0], jnp.int32)
    idxc = c * CH + jax.lax.broadcasted_iota(jnp.int32, (1, CH), 1)
    mk = (bits > tau) | ((bits == tau) & (idxc <= mlow))
    t_c = t_ref[:, pl.ds(c * CH, CH)]
    nl_c = nl_ref[:, pl.ds(c * CH, CH)]
    w = jnp.where(mk, jnp.exp(nl_c * t_c), 0.0)
    part = jnp.dot(w, q_ref[...], preferred_element_type=jnp.float32)

    @pl.when(c == 0)
    def _init():
        out_ref[0] = part

    @pl.when(c > 0)
    def _acc():
        out_ref[0] = out_ref[0] + part


def kernel(time, negative_lambdas, quantities, top_k, min_tokens_to_keep):
    t3 = time.reshape(S, 1, L)
    nll3 = negative_lambdas[:, H - 1, :].reshape(S, 1, L)
    q3 = quantities.reshape(S, L, D)
    rel = pl.pallas_call(
        _rel_body,
        grid=(S,),
        in_specs=[
            pl.BlockSpec((1, 1, L), lambda s: (s, 0, 0)),
            pl.BlockSpec((1, 1, L), lambda s: (s, 0, 0)),
            pl.BlockSpec((1, L, D), lambda s: (s, 0, 0)),
        ],
        out_specs=pl.BlockSpec((1, 1, L), lambda s: (s, 0, 0)),
        out_shape=jax.ShapeDtypeStruct((S, 1, L), jnp.float32),
    )(t3, nll3, q3)

    nl2 = negative_lambdas.reshape(H, N)
    q2 = quantities.reshape(N, D)
    return pl.pallas_call(
        _sel_mm_body,
        grid=(NCH,),
        in_specs=[
            pl.BlockSpec((S, L), lambda c: (0, 0)),
            pl.BlockSpec((1, 1, CH), lambda c: (c, 0, 0)),
            pl.BlockSpec((1, N), lambda c: (0, 0)),
            pl.BlockSpec((H, N), lambda c: (0, 0)),
            pl.BlockSpec((CH, D), lambda c: (c, 0)),
        ],
        out_specs=pl.BlockSpec((1, H, D), lambda c: (0, 0, 0)),
        out_shape=jax.ShapeDtypeStruct((1, H, D), jnp.float32),
        scratch_shapes=[pltpu.SMEM((2,), jnp.int32)],
    )(rel.reshape(S, L), rel.reshape(NCH, 1, CH), time, nl2, q2)


# speedup vs baseline: 3.5532x; 1.4816x over previous
import jax
import jax.numpy as jnp
from jax.experimental import pallas as pl
from jax.experimental.pallas import tpu as pltpu

N = 32768
H = 16
D = 64
S = 8
L = N // S
K = min(N, max(4096, 16))
CH = 2048
NCH = N // CH
RCH = 512


def _rel_body(t_ref, nll_ref, q_ref, rel_ref):
    ones = jnp.ones((1, D), jnp.float32)
    parts = []
    for u in range(L // RCH):
        q_u = q_ref[0, u * RCH:(u + 1) * RCH, :]
        sq = q_u * q_u
        parts.append(jax.lax.dot_general(
            ones, sq, (((1,), (1,)), ((), ())),
            precision=jax.lax.Precision.HIGHEST,
            preferred_element_type=jnp.float32))
    mag2 = jnp.concatenate(parts, axis=1)
    rel_ref[0] = jnp.sqrt(mag2) * jnp.exp(nll_ref[0] * t_ref[0])


def _sel_mm_body(rels_ref, relc_ref, t_ref, nl_ref, q_ref, out_ref, sc_ref):
    c = pl.program_id(0)

    @pl.when(c == 0)
    def _search():
        rel = rels_ref[...]
        relbits = jax.lax.bitcast_convert_type(rel, jnp.int32)

        def vsearch(i, lo):
            cand = lo | (jnp.int32(1) << (jnp.int32(30) - i))
            cnt = jnp.sum((relbits >= cand).astype(jnp.int32))
            return jnp.where(cnt >= K, cand, lo)

        taubits = jax.lax.fori_loop(0, 31, vsearch, jnp.int32(0))

        cnt_gt = jnp.sum((relbits > taubits).astype(jnp.int32))
        need = K - cnt_gt
        eq = relbits == taubits
        idx = (jax.lax.broadcasted_iota(jnp.int32, (S, L), 0) * L
               + jax.lax.broadcasted_iota(jnp.int32, (S, L), 1))

        def isearch(i, m):
            cand = m | (jnp.int32(1) << (jnp.int32(14) - i))
            cnt = jnp.sum(jnp.where(eq & (idx < cand), 1, 0))
            return jnp.where(cnt < need, cand, m)

        mlow = jax.lax.fori_loop(0, 15, isearch, jnp.int32(0))
        sc_ref[0] = taubits
        sc_ref[1] = mlow

    tau = sc_ref[0]
    mlow = sc_ref[1]
    bits = jax.lax.bitcast_convert_type(relc_ref[0], jnp.int32)
    idxc = c * CH + jax.lax.broadcasted_iota(jnp.int32, (1, CH), 1)
    mk = (bits > tau) | ((bits == tau) & (idxc <= mlow))
    t_c = t_ref[:, pl.ds(c * CH, CH)]
    nl_c = nl_ref[:, pl.ds(c * CH, CH)]
    w = jnp.where(mk, jnp.exp(nl_c * t_c), 0.0)
    part = jnp.dot(w, q_ref[...], preferred_element_type=jnp.float32)

    @pl.when(c == 0)
    def _init():
        out_ref[0] = part

    @pl.when(c > 0)
    def _acc():
        out_ref[0] = out_ref[0] + part


def kernel(time, negative_lambdas, quantities, top_k, min_tokens_to_keep):
    t3 = time.reshape(S, 1, L)
    nll3 = negative_lambdas[:, H - 1, :].reshape(S, 1, L)
    q3 = quantities.reshape(S, L, D)
    rel = pl.pallas_call(
        _rel_body,
        grid=(S,),
        in_specs=[
            pl.BlockSpec((1, 1, L), lambda s: (s, 0, 0)),
            pl.BlockSpec((1, 1, L), lambda s: (s, 0, 0)),
            pl.BlockSpec((1, L, D), lambda s: (s, 0, 0)),
        ],
        out_specs=pl.BlockSpec((1, 1, L), lambda s: (s, 0, 0)),
        out_shape=jax.ShapeDtypeStruct((S, 1, L), jnp.float32),
    )(t3, nll3, q3)

    nl2 = negative_lambdas.reshape(H, N)
    q2 = quantities.reshape(N, D)
    return pl.pallas_call(
        _sel_mm_body,
        grid=(NCH,),
        in_specs=[
            pl.BlockSpec((S, L), lambda c: (0, 0)),
            pl.BlockSpec((1, 1, CH), lambda c: (c, 0, 0)),
            pl.BlockSpec((1, N), lambda c: (0, 0)),
            pl.BlockSpec((H, N), lambda c: (0, 0)),
            pl.BlockSpec((CH, D), lambda c: (c, 0)),
        ],
        out_specs=pl.BlockSpec((1, H, D), lambda c: (0, 0, 0)),
        out_shape=jax.ShapeDtypeStruct((1, H, D), jnp.float32),
        scratch_shapes=[pltpu.SMEM((2,), jnp.int32)],
    )(rel.reshape(S, L), rel.reshape(NCH, 1, CH), time, nl2, q2)
